# Initial kernel scaffold; baseline (speedup 1.0000x reference)
import jax
import jax.numpy as jnp
from jax.experimental import pallas as pl


def kernel(x, perm, inv_perm, group_scales):
    raise NotImplementedError("write your pallas kernel here")


# speedup vs baseline: 6.7368x; 6.7368x over previous
import jax
import jax.numpy as jnp
from jax.experimental import pallas as pl

_B, _S, _D = 4, 8192, 2048
_G = 64
_NG = _D // _G
_QMAX = 127.0

_ROWS = 512


def _quant_body(inv_ref, gs_ref, x_ref, o_ref):
    g = (inv_ref[...] // _G).astype(jnp.int32)
    ids = jax.lax.broadcasted_iota(jnp.int32, (_NG, _D), 0)
    onehot = (g == ids)
    s = jnp.sum(jnp.where(onehot, gs_ref[...], 0.0), axis=0, keepdims=True)
    xv = x_ref[...]
    q = jnp.clip(jnp.round(xv / s), -_QMAX, _QMAX)
    o_ref[...] = q * s


def kernel(x, perm, inv_perm, group_scales):
    del perm
    xf = x.reshape(_B * _S, _D)
    inv2 = inv_perm.astype(jnp.int32).reshape(1, _D)
    gs2 = group_scales.astype(jnp.float32).reshape(_NG, 1)
    grid = (xf.shape[0] // _ROWS,)
    out = pl.pallas_call(
        _quant_body,
        grid=grid,
        in_specs=[
            pl.BlockSpec((1, _D), lambda i: (0, 0)),
            pl.BlockSpec((_NG, 1), lambda i: (0, 0)),
            pl.BlockSpec((_ROWS, _D), lambda i: (i, 0)),
        ],
        out_specs=pl.BlockSpec((_ROWS, _D), lambda i: (i, 0)),
        out_shape=jax.ShapeDtypeStruct(xf.shape, x.dtype),
    )(inv2, gs2, xf)
    return out.reshape(x.shape)


# speedup vs baseline: 6.9020x; 1.0245x over previous
import jax
import jax.numpy as jnp
from jax.experimental import pallas as pl

_B, _S, _D = 4, 8192, 2048
_G = 64
_NG = _D // _G
_QMAX = 127.0

_ROWS = 1024


def _quant_body(inv_ref, gs_ref, x_ref, o_ref):
    g = (inv_ref[...] // _G).astype(jnp.int32)
    ids = jax.lax.broadcasted_iota(jnp.int32, (_NG, _D), 0)
    onehot = (g == ids)
    s = jnp.sum(jnp.where(onehot, gs_ref[...], 0.0), axis=0, keepdims=True)
    xv = x_ref[...]
    q = jnp.clip(jnp.round(xv / s), -_QMAX, _QMAX)
    o_ref[...] = q * s


def kernel(x, perm, inv_perm, group_scales):
    del perm
    xf = x.reshape(_B * _S, _D)
    inv2 = inv_perm.astype(jnp.int32).reshape(1, _D)
    gs2 = group_scales.astype(jnp.float32).reshape(_NG, 1)
    grid = (xf.shape[0] // _ROWS,)
    out = pl.pallas_call(
        _quant_body,
        grid=grid,
        in_specs=[
            pl.BlockSpec((1, _D), lambda i: (0, 0)),
            pl.BlockSpec((_NG, 1), lambda i: (0, 0)),
            pl.BlockSpec((_ROWS, _D), lambda i: (i, 0)),
        ],
        out_specs=pl.BlockSpec((_ROWS, _D), lambda i: (i, 0)),
        out_shape=jax.ShapeDtypeStruct(xf.shape, x.dtype),
    )(inv2, gs2, xf)
    return out.reshape(x.shape)
